# Initial kernel scaffold; baseline (speedup 1.0000x reference)
import jax
import jax.numpy as jnp
from jax.experimental import pallas as pl


def kernel(x, edge_index, W1, b1, W2, b2, W3, b3):
    raise NotImplementedError("write your pallas kernel here")


# speedup vs baseline: 3.9570x; 3.9570x over previous
import functools

import jax
import jax.numpy as jnp
from jax import lax
from jax.experimental import pallas as pl
from jax.experimental.pallas import tpu as pltpu
from jax.experimental.pallas import tpu_sc as plsc

NC = 2
NS = 16
KB = 128
BN = 512
ET = 1024


def _ceil_to(a, m):
    return (a + m - 1) // m * m


def _gather_body(e_pad, w, u_hbm, idx_hbm, g_hbm, idx_v, rbuf, sem):
    wid = lax.axis_index("s") * NC + lax.axis_index("c")
    ew = e_pad // (NC * NS)
    base = wid * ew
    pltpu.sync_copy(idx_hbm.at[pl.ds(base, ew)], idx_v)

    @pl.loop(0, ew // KB)
    def _(b):
        pltpu.async_copy(
            u_hbm.at[idx_v.at[pl.ds(b * KB, KB)]], rbuf, sem
        ).wait()
        pltpu.sync_copy(rbuf, g_hbm.at[pl.ds(base + b * KB, KB)])


def _make_gather(n_pad, e_pad, w):
    mesh = plsc.VectorSubcoreMesh(
        core_axis_name="c", subcore_axis_name="s", num_cores=NC, num_subcores=NS
    )
    return pl.kernel(
        functools.partial(_gather_body, e_pad, w),
        out_type=jax.ShapeDtypeStruct((e_pad, w), jnp.float32),
        mesh=mesh,
        scratch_types=[
            pltpu.VMEM((e_pad // (NC * NS),), jnp.int32),
            pltpu.VMEM((KB, w), jnp.float32),
            pltpu.SemaphoreType.DMA,
        ],
    )


def _agg_tiles(i, rp_ref, dst_hbm, g_hbm, dst_v, gbuf, acc_ref, sem_d, sem_g,
               with_g=True):
    lo = rp_ref[i]
    hi = rp_ref[i + 1]
    t0 = lo // ET
    t1 = lax.div(hi + ET - 1, ET)
    nbase = i * BN

    def tile_body(t, carry):
        cd = pltpu.make_async_copy(dst_hbm.at[t], dst_v, sem_d)
        cd.start()
        if with_g:
            cg = pltpu.make_async_copy(
                g_hbm.at[pl.ds(t * ET, ET)], gbuf, sem_g
            )
            cg.start()
        cd.wait()
        dstb = jnp.broadcast_to(dst_v[...].reshape(1, ET), (BN, ET))
        rows = (
            lax.broadcasted_iota(jnp.int32, (BN, ET), 0) + nbase
        )
        sel = jnp.where(rows == dstb, 1.0, 0.0).astype(jnp.float32)
        if with_g:
            cg.wait()
            acc_ref[...] += jnp.dot(
                sel, gbuf[...], preferred_element_type=jnp.float32
            )
        else:
            acc_ref[...] += jnp.broadcast_to(
                jnp.sum(sel, axis=1, keepdims=True), acc_ref.shape
            )
        return carry

    lax.fori_loop(t0, t1, tile_body, 0)


def _deg_body(rp_ref, dst_hbm, x_ref, s_ref, u0_ref, dst_v, acc_ref, sem_d):
    i = pl.program_id(0)
    acc_ref[...] = jnp.zeros_like(acc_ref)
    _agg_tiles(i, rp_ref, dst_hbm, None, dst_v, None, acc_ref, sem_d, None,
               with_g=False)
    s = lax.rsqrt(acc_ref[:, 0:1] + 1.0)
    s_ref[...] = jnp.broadcast_to(s, s_ref.shape)
    u0_ref[...] = s * x_ref[...]


def _l1_body(rp_ref, dst_hbm, g_hbm, up_ref, s_ref, w_ref, b_ref, o_ref,
             dst_v, gbuf, acc_ref, sem_d, sem_g):
    i = pl.program_id(0)
    acc_ref[...] = jnp.zeros_like(acc_ref)
    _agg_tiles(i, rp_ref, dst_hbm, g_hbm, dst_v, gbuf, acc_ref, sem_d, sem_g)
    s = s_ref[:, 0:1]
    a = acc_ref[...] + up_ref[...]
    y = jnp.maximum(
        jnp.dot(s * a, w_ref[...], preferred_element_type=jnp.float32)
        + b_ref[...],
        0.0,
    )
    o_ref[...] = s * y


def _l2_body(rp_ref, dst_hbm, g_hbm, up_ref, s_ref, w2_ref, b2_ref, w3_ref,
             o_ref, dst_v, gbuf, acc_ref, sem_d, sem_g):
    i = pl.program_id(0)
    acc_ref[...] = jnp.zeros_like(acc_ref)
    _agg_tiles(i, rp_ref, dst_hbm, g_hbm, dst_v, gbuf, acc_ref, sem_d, sem_g)
    s = s_ref[:, 0:1]
    a = acc_ref[...] + up_ref[...]
    y = jnp.maximum(
        jnp.dot(s * a, w2_ref[...], preferred_element_type=jnp.float32)
        + b2_ref[...],
        0.0,
    )
    o_ref[...] = jnp.dot(
        s * y, w3_ref[...], preferred_element_type=jnp.float32
    )


def _l3_body(rp_ref, dst_hbm, g_hbm, up_ref, s_ref, b_ref, o_ref,
             dst_v, gbuf, acc_ref, sem_d, sem_g):
    i = pl.program_id(0)
    acc_ref[...] = jnp.zeros_like(acc_ref)
    _agg_tiles(i, rp_ref, dst_hbm, g_hbm, dst_v, gbuf, acc_ref, sem_d, sem_g)
    s = s_ref[:, 0:1]
    o_ref[...] = s * (acc_ref[...] + up_ref[...]) + b_ref[...]


def _smem_spec():
    return pl.BlockSpec(memory_space=pltpu.SMEM)


def _any_spec():
    return pl.BlockSpec(memory_space=pl.ANY)


def _row_spec(w):
    return pl.BlockSpec((BN, w), lambda i: (i, 0))


def _full_spec(a, b):
    return pl.BlockSpec((a, b), lambda i: (0, 0))


def kernel(x, edge_index, W1, b1, W2, b2, W3, b3):
    N, d_in = x.shape
    d_hid = W1.shape[1]
    d_out = W3.shape[1]
    E = edge_index.shape[1]
    n_pad = _ceil_to(N, BN)
    e_pad = _ceil_to(E, max(NC * NS * KB, ET))
    nb = n_pad // BN

    src = jnp.concatenate(
        [edge_index[0].astype(jnp.int32), jnp.full((e_pad - E,), N, jnp.int32)]
    )
    dst = jnp.concatenate(
        [edge_index[1].astype(jnp.int32), jnp.full((e_pad - E,), N, jnp.int32)]
    )
    perm = jnp.argsort(dst)
    dst_s = dst[perm]
    src_s = src[perm]
    rowptr = jnp.searchsorted(
        dst_s, jnp.arange(0, n_pad + 1, BN, dtype=jnp.int32)
    ).astype(jnp.int32)
    dst3 = dst_s.reshape(e_pad // ET, 1, ET)

    x_p = jnp.pad(x, ((0, n_pad - N), (0, 0)))
    b1_2 = b1.reshape(1, -1)
    b2_2 = b2.reshape(1, -1)
    b3_2 = b3.reshape(1, -1)

    grid = (nb,)
    vmem_i = lambda: pltpu.VMEM((1, ET), jnp.int32)
    sem = pltpu.SemaphoreType.DMA

    s_full, u0 = pl.pallas_call(
        _deg_body,
        grid=grid,
        in_specs=[_smem_spec(), _any_spec(), _row_spec(d_in)],
        out_specs=[_row_spec(128), _row_spec(d_in)],
        out_shape=[
            jax.ShapeDtypeStruct((n_pad, 128), jnp.float32),
            jax.ShapeDtypeStruct((n_pad, d_in), jnp.float32),
        ],
        scratch_shapes=[vmem_i(), pltpu.VMEM((BN, 128), jnp.float32), sem],
    )(rowptr, dst3, x_p)

    g1 = _make_gather(n_pad, e_pad, d_in)(u0, src_s)
    u1 = pl.pallas_call(
        _l1_body,
        grid=grid,
        in_specs=[_smem_spec(), _any_spec(), _any_spec(), _row_spec(d_in),
                  _row_spec(128), _full_spec(d_in, d_hid),
                  _full_spec(1, d_hid)],
        out_specs=_row_spec(d_hid),
        out_shape=jax.ShapeDtypeStruct((n_pad, d_hid), jnp.float32),
        scratch_shapes=[vmem_i(), pltpu.VMEM((ET, d_in), jnp.float32),
                        pltpu.VMEM((BN, d_in), jnp.float32), sem, sem],
    )(rowptr, dst3, g1, u0, s_full, W1, b1_2)

    g2 = _make_gather(n_pad, e_pad, d_hid)(u1, src_s)
    t2 = pl.pallas_call(
        _l2_body,
        grid=grid,
        in_specs=[_smem_spec(), _any_spec(), _any_spec(), _row_spec(d_hid),
                  _row_spec(128), _full_spec(d_hid, d_hid),
                  _full_spec(1, d_hid), _full_spec(d_hid, d_out)],
        out_specs=_row_spec(d_out),
        out_shape=jax.ShapeDtypeStruct((n_pad, d_out), jnp.float32),
        scratch_shapes=[vmem_i(), pltpu.VMEM((ET, d_hid), jnp.float32),
                        pltpu.VMEM((BN, d_hid), jnp.float32), sem, sem],
    )(rowptr, dst3, g2, u1, s_full, W2, b2_2, W3)

    g3 = _make_gather(n_pad, e_pad, d_out)(t2, src_s)
    out = pl.pallas_call(
        _l3_body,
        grid=grid,
        in_specs=[_smem_spec(), _any_spec(), _any_spec(), _row_spec(d_out),
                  _row_spec(128), _full_spec(1, d_out)],
        out_specs=_row_spec(d_out),
        out_shape=jax.ShapeDtypeStruct((n_pad, d_out), jnp.float32),
        scratch_shapes=[vmem_i(), pltpu.VMEM((ET, d_out), jnp.float32),
                        pltpu.VMEM((BN, d_out), jnp.float32), sem, sem],
    )(rowptr, dst3, g3, t2, s_full, b3_2)

    return out[:N]


# speedup vs baseline: 3.9846x; 1.0070x over previous
import functools

import jax
import jax.numpy as jnp
from jax import lax
from jax.experimental import pallas as pl
from jax.experimental.pallas import tpu as pltpu
from jax.experimental.pallas import tpu_sc as plsc

NC = 2
NS = 16
KB = 128
BN = 512
ET = 1024


def _ceil_to(a, m):
    return (a + m - 1) // m * m


def _gather_body(e_pad, w, kb, u_hbm, idx_hbm, g_hbm, idx_v, rb0, rb1,
                 sem0, sem1):
    wid = lax.axis_index("s") * NC + lax.axis_index("c")
    ew = e_pad // (NC * NS)
    base = wid * ew
    pltpu.sync_copy(idx_hbm.at[pl.ds(base, ew)], idx_v)

    @pl.loop(0, ew // (2 * kb))
    def _(g):
        b0 = 2 * g * kb
        b1 = b0 + kb
        c0 = pltpu.async_copy(u_hbm.at[idx_v.at[pl.ds(b0, kb)]], rb0, sem0)
        c1 = pltpu.async_copy(u_hbm.at[idx_v.at[pl.ds(b1, kb)]], rb1, sem1)
        c0.wait()
        pltpu.sync_copy(rb0, g_hbm.at[pl.ds(base + b0, kb)])
        c1.wait()
        pltpu.sync_copy(rb1, g_hbm.at[pl.ds(base + b1, kb)])


def _make_gather(n_pad, e_pad, w):
    kb = KB
    while 2 * kb * w * 4 > 384 * 1024:
        kb //= 2
    mesh = plsc.VectorSubcoreMesh(
        core_axis_name="c", subcore_axis_name="s", num_cores=NC, num_subcores=NS
    )
    return pl.kernel(
        functools.partial(_gather_body, e_pad, w, kb),
        out_type=jax.ShapeDtypeStruct((e_pad, w), jnp.float32),
        mesh=mesh,
        scratch_types=[
            pltpu.VMEM((e_pad // (NC * NS),), jnp.int32),
            pltpu.VMEM((kb, w), jnp.float32),
            pltpu.VMEM((kb, w), jnp.float32),
            pltpu.SemaphoreType.DMA,
            pltpu.SemaphoreType.DMA,
        ],
    )


def _agg_tiles(i, rp_ref, dst_hbm, g_hbm, dst_v, gbuf, acc_ref, sem_d, sem_g,
               with_g=True):
    lo = rp_ref[i]
    hi = rp_ref[i + 1]
    t0 = lo // ET
    t1 = lax.div(hi + ET - 1, ET)
    nbase = i * BN

    def tile_body(t, carry):
        cd = pltpu.make_async_copy(dst_hbm.at[t], dst_v, sem_d)
        cd.start()
        if with_g:
            cg = pltpu.make_async_copy(
                g_hbm.at[pl.ds(t * ET, ET)], gbuf, sem_g
            )
            cg.start()
        cd.wait()
        dstb = jnp.broadcast_to(dst_v[...].reshape(1, ET), (BN, ET))
        rows = (
            lax.broadcasted_iota(jnp.int32, (BN, ET), 0) + nbase
        )
        sel = jnp.where(rows == dstb, 1.0, 0.0).astype(jnp.float32)
        if with_g:
            cg.wait()
            acc_ref[...] += jnp.dot(
                sel, gbuf[...], preferred_element_type=jnp.float32
            )
        else:
            acc_ref[...] += jnp.broadcast_to(
                jnp.sum(sel, axis=1, keepdims=True), acc_ref.shape
            )
        return carry

    lax.fori_loop(t0, t1, tile_body, 0)


def _deg_body(rp_ref, dst_hbm, x_ref, s_ref, u0_ref, dst_v, acc_ref, sem_d):
    i = pl.program_id(0)
    acc_ref[...] = jnp.zeros_like(acc_ref)
    _agg_tiles(i, rp_ref, dst_hbm, None, dst_v, None, acc_ref, sem_d, None,
               with_g=False)
    s = lax.rsqrt(acc_ref[:, 0:1] + 1.0)
    s_ref[...] = jnp.broadcast_to(s, s_ref.shape)
    u0_ref[...] = s * x_ref[...]


def _l1_body(rp_ref, dst_hbm, g_hbm, up_ref, s_ref, w_ref, b_ref, o_ref,
             dst_v, gbuf, acc_ref, sem_d, sem_g):
    i = pl.program_id(0)
    acc_ref[...] = jnp.zeros_like(acc_ref)
    _agg_tiles(i, rp_ref, dst_hbm, g_hbm, dst_v, gbuf, acc_ref, sem_d, sem_g)
    s = s_ref[:, 0:1]
    a = acc_ref[...] + up_ref[...]
    y = jnp.maximum(
        jnp.dot(s * a, w_ref[...], preferred_element_type=jnp.float32)
        + b_ref[...],
        0.0,
    )
    o_ref[...] = s * y


def _l2_body(rp_ref, dst_hbm, g_hbm, up_ref, s_ref, w2_ref, b2_ref, w3_ref,
             o_ref, dst_v, gbuf, acc_ref, sem_d, sem_g):
    i = pl.program_id(0)
    acc_ref[...] = jnp.zeros_like(acc_ref)
    _agg_tiles(i, rp_ref, dst_hbm, g_hbm, dst_v, gbuf, acc_ref, sem_d, sem_g)
    s = s_ref[:, 0:1]
    a = acc_ref[...] + up_ref[...]
    y = jnp.maximum(
        jnp.dot(s * a, w2_ref[...], preferred_element_type=jnp.float32)
        + b2_ref[...],
        0.0,
    )
    o_ref[...] = jnp.dot(
        s * y, w3_ref[...], preferred_element_type=jnp.float32
    )


def _l3_body(rp_ref, dst_hbm, g_hbm, up_ref, s_ref, b_ref, o_ref,
             dst_v, gbuf, acc_ref, sem_d, sem_g):
    i = pl.program_id(0)
    acc_ref[...] = jnp.zeros_like(acc_ref)
    _agg_tiles(i, rp_ref, dst_hbm, g_hbm, dst_v, gbuf, acc_ref, sem_d, sem_g)
    s = s_ref[:, 0:1]
    o_ref[...] = s * (acc_ref[...] + up_ref[...]) + b_ref[...]


def _smem_spec():
    return pl.BlockSpec(memory_space=pltpu.SMEM)


def _any_spec():
    return pl.BlockSpec(memory_space=pl.ANY)


def _row_spec(w):
    return pl.BlockSpec((BN, w), lambda i: (i, 0))


def _full_spec(a, b):
    return pl.BlockSpec((a, b), lambda i: (0, 0))


def kernel(x, edge_index, W1, b1, W2, b2, W3, b3):
    N, d_in = x.shape
    d_hid = W1.shape[1]
    d_out = W3.shape[1]
    E = edge_index.shape[1]
    n_pad = _ceil_to(N, BN)
    e_pad = _ceil_to(E, max(NC * NS * KB, ET))
    nb = n_pad // BN

    src = jnp.concatenate(
        [edge_index[0].astype(jnp.int32), jnp.full((e_pad - E,), N, jnp.int32)]
    )
    dst = jnp.concatenate(
        [edge_index[1].astype(jnp.int32), jnp.full((e_pad - E,), N, jnp.int32)]
    )
    perm = jnp.argsort(dst)
    dst_s = dst[perm]
    src_s = src[perm]
    rowptr = jnp.searchsorted(
        dst_s, jnp.arange(0, n_pad + 1, BN, dtype=jnp.int32)
    ).astype(jnp.int32)
    dst3 = dst_s.reshape(e_pad // ET, 1, ET)

    x_p = jnp.pad(x, ((0, n_pad - N), (0, 0)))
    b1_2 = b1.reshape(1, -1)
    b2_2 = b2.reshape(1, -1)
    b3_2 = b3.reshape(1, -1)

    grid = (nb,)
    vmem_i = lambda: pltpu.VMEM((1, ET), jnp.int32)
    sem = pltpu.SemaphoreType.DMA

    s_full, u0 = pl.pallas_call(
        _deg_body,
        grid=grid,
        in_specs=[_smem_spec(), _any_spec(), _row_spec(d_in)],
        out_specs=[_row_spec(128), _row_spec(d_in)],
        out_shape=[
            jax.ShapeDtypeStruct((n_pad, 128), jnp.float32),
            jax.ShapeDtypeStruct((n_pad, d_in), jnp.float32),
        ],
        scratch_shapes=[vmem_i(), pltpu.VMEM((BN, 128), jnp.float32), sem],
    )(rowptr, dst3, x_p)

    g1 = _make_gather(n_pad, e_pad, d_in)(u0, src_s)
    u1 = pl.pallas_call(
        _l1_body,
        grid=grid,
        in_specs=[_smem_spec(), _any_spec(), _any_spec(), _row_spec(d_in),
                  _row_spec(128), _full_spec(d_in, d_hid),
                  _full_spec(1, d_hid)],
        out_specs=_row_spec(d_hid),
        out_shape=jax.ShapeDtypeStruct((n_pad, d_hid), jnp.float32),
        scratch_shapes=[vmem_i(), pltpu.VMEM((ET, d_in), jnp.float32),
                        pltpu.VMEM((BN, d_in), jnp.float32), sem, sem],
    )(rowptr, dst3, g1, u0, s_full, W1, b1_2)

    g2 = _make_gather(n_pad, e_pad, d_hid)(u1, src_s)
    t2 = pl.pallas_call(
        _l2_body,
        grid=grid,
        in_specs=[_smem_spec(), _any_spec(), _any_spec(), _row_spec(d_hid),
                  _row_spec(128), _full_spec(d_hid, d_hid),
                  _full_spec(1, d_hid), _full_spec(d_hid, d_out)],
        out_specs=_row_spec(d_out),
        out_shape=jax.ShapeDtypeStruct((n_pad, d_out), jnp.float32),
        scratch_shapes=[vmem_i(), pltpu.VMEM((ET, d_hid), jnp.float32),
                        pltpu.VMEM((BN, d_hid), jnp.float32), sem, sem],
    )(rowptr, dst3, g2, u1, s_full, W2, b2_2, W3)

    g3 = _make_gather(n_pad, e_pad, d_out)(t2, src_s)
    out = pl.pallas_call(
        _l3_body,
        grid=grid,
        in_specs=[_smem_spec(), _any_spec(), _any_spec(), _row_spec(d_out),
                  _row_spec(128), _full_spec(1, d_out)],
        out_specs=_row_spec(d_out),
        out_shape=jax.ShapeDtypeStruct((n_pad, d_out), jnp.float32),
        scratch_shapes=[vmem_i(), pltpu.VMEM((ET, d_out), jnp.float32),
                        pltpu.VMEM((BN, d_out), jnp.float32), sem, sem],
    )(rowptr, dst3, g3, t2, s_full, b3_2)

    return out[:N]
